# baseline (device time: 39875 ns/iter reference)
import jax
import jax.numpy as jnp
from jax import lax
from jax.experimental import pallas as pl
from jax.experimental.pallas import tpu as pltpu

N_DEV = 8
M = 512

RS_MASKS = (4, 3, 1)
AG_MASKS = (1, 3, 4)


def kernel(dy, W):
    m, k = dy.shape
    assert W.shape == (m, k)

    def body(dy_ref, w_ref, out_ref,
             acc1, acc2, g0, g1, g2,
             rs_send0, rs_send1, rs_send2,
             rs_recv0, rs_recv1, rs_recv2,
             ag_recv0, ag_recv1, ag_recv2,
             send_sems, recv_sems):
        p = lax.axis_index("i")

        barrier_sem = pltpu.get_barrier_semaphore()
        for mask in (1, 3, 4):
            pl.semaphore_signal(
                barrier_sem, inc=1,
                device_id=(jnp.bitwise_xor(p, mask),),
                device_id_type=pl.DeviceIdType.MESH,
            )
        pl.semaphore_wait(barrier_sem, 3)

        out_ref[...] = lax.dot_general(
            dy_ref[...], w_ref[...],
            dimension_numbers=(((1,), (1,)), ((), ())),
            preferred_element_type=jnp.float32,
        )

        def exchange(e, mask, send_buf, recv_buf):
            q = jnp.bitwise_xor(p, mask)
            rdma = pltpu.make_async_remote_copy(
                src_ref=send_buf,
                dst_ref=recv_buf,
                send_sem=send_sems.at[e],
                recv_sem=recv_sems.at[e],
                device_id=(q,),
                device_id_type=pl.DeviceIdType.MESH,
            )
            rdma.start()
            rdma.wait()

        low_z = p < jnp.bitwise_xor(p, RS_MASKS[0])
        rs_send0[...] = jnp.where(low_z, out_ref[256:, :], out_ref[:256, :])
        exchange(0, RS_MASKS[0], rs_send0, rs_recv0)
        acc1[...] = (
            jnp.where(low_z, out_ref[:256, :], out_ref[256:, :])
            + rs_recv0[...]
        )

        low_y = p < jnp.bitwise_xor(p, RS_MASKS[1])
        rs_send1[...] = jnp.where(low_y, acc1[128:, :], acc1[:128, :])
        exchange(1, RS_MASKS[1], rs_send1, rs_recv1)
        acc2[...] = (
            jnp.where(low_y, acc1[:128, :], acc1[128:, :]) + rs_recv1[...]
        )

        low_x = p < jnp.bitwise_xor(p, RS_MASKS[2])
        rs_send2[...] = jnp.where(low_x, acc2[64:, :], acc2[:64, :])
        exchange(2, RS_MASKS[2], rs_send2, rs_recv2)
        g0[...] = jnp.where(low_x, acc2[:64, :], acc2[64:, :]) + rs_recv2[...]

        exchange(3, AG_MASKS[0], g0, ag_recv0)
        g1[...] = jnp.where(
            low_x,
            jnp.concatenate([g0[...], ag_recv0[...]], axis=0),
            jnp.concatenate([ag_recv0[...], g0[...]], axis=0),
        )

        exchange(4, AG_MASKS[1], g1, ag_recv1)
        g2[...] = jnp.where(
            low_y,
            jnp.concatenate([g1[...], ag_recv1[...]], axis=0),
            jnp.concatenate([ag_recv1[...], g1[...]], axis=0),
        )

        exchange(5, AG_MASKS[2], g2, ag_recv2)
        out_ref[...] = jnp.where(
            low_z,
            jnp.concatenate([g2[...], ag_recv2[...]], axis=0),
            jnp.concatenate([ag_recv2[...], g2[...]], axis=0),
        )

    return pl.pallas_call(
        body,
        out_shape=jax.ShapeDtypeStruct((M, M), jnp.float32),
        in_specs=[
            pl.BlockSpec(memory_space=pltpu.VMEM),
            pl.BlockSpec(memory_space=pltpu.VMEM),
        ],
        out_specs=pl.BlockSpec(memory_space=pltpu.VMEM),
        scratch_shapes=[
            pltpu.VMEM((256, M), jnp.float32),
            pltpu.VMEM((128, M), jnp.float32),
            pltpu.VMEM((64, M), jnp.float32),
            pltpu.VMEM((128, M), jnp.float32),
            pltpu.VMEM((256, M), jnp.float32),
            pltpu.VMEM((256, M), jnp.float32),
            pltpu.VMEM((128, M), jnp.float32),
            pltpu.VMEM((64, M), jnp.float32),
            pltpu.VMEM((256, M), jnp.float32),
            pltpu.VMEM((128, M), jnp.float32),
            pltpu.VMEM((64, M), jnp.float32),
            pltpu.VMEM((64, M), jnp.float32),
            pltpu.VMEM((128, M), jnp.float32),
            pltpu.VMEM((256, M), jnp.float32),
            pltpu.SemaphoreType.DMA((6,)),
            pltpu.SemaphoreType.DMA((6,)),
        ],
        compiler_params=pltpu.CompilerParams(collective_id=0),
    )(dy, W)


# device time: 28740 ns/iter; 1.3874x vs baseline; 1.3874x over previous
import jax
import jax.numpy as jnp
from jax import lax
from jax.experimental import pallas as pl
from jax.experimental.pallas import tpu as pltpu

M = 512

SPLITS = ((0, 192), (192, 192), (384, 128))
RS_MASKS = ((4, 3, 1), (3, 1, 4), (1, 4, 3))

def _block_scratch(r):
    h0, h1, h2 = r // 2, r // 4, r // 8
    rows = [h0, h1, h2, h1, h0, h0, h1, h2, h0, h1, h2, h2, h1, h0]
    return [pltpu.VMEM((h, M), jnp.float32) for h in rows]


N_PER_B = 14


def kernel(dy, W):
    m, k = dy.shape
    assert W.shape == (m, k) and m == M

    def body(dy_ref, w_ref, out_ref, *scratch):
        send_sems, recv_sems = scratch[-2], scratch[-1]
        bufs = [scratch[b * N_PER_B:(b + 1) * N_PER_B] for b in range(3)]

        p = lax.axis_index("i")

        barrier_sem = pltpu.get_barrier_semaphore()
        for mask in (1, 3, 4):
            pl.semaphore_signal(
                barrier_sem, inc=1,
                device_id=(jnp.bitwise_xor(p, mask),),
                device_id_type=pl.DeviceIdType.MESH,
            )
        pl.semaphore_wait(barrier_sem, 3)

        out_ref[...] = lax.dot_general(
            dy_ref[...], w_ref[...],
            dimension_numbers=(((1,), (1,)), ((), ())),
            preferred_element_type=jnp.float32,
        )

        lows = [
            [(p & 4) == 0, (p & 2) == 0, (p & 1) == 0],
            [(p & 2) == 0, (p & 1) == 0, (p & 4) == 0],
            [((p ^ (p >> 1)) & 1) == 0, (p & 4) == 0, (p & 2) == 0],
        ]

        def start(e, b, s, src, dst):
            rdma = pltpu.make_async_remote_copy(
                src_ref=src, dst_ref=dst,
                send_sem=send_sems.at[e],
                recv_sem=recv_sems.at[e],
                device_id=(jnp.bitwise_xor(p, RS_MASKS[b][s]),),
                device_id_type=pl.DeviceIdType.MESH,
            )
            rdma.start()
            return rdma

        rdmas = [None] * 3
        for b, (o, r) in enumerate(SPLITS):
            acc1, acc2, g0, g1, g2, s0, s1, s2, rr0, rr1, rr2, a0, a1, a2 = bufs[b]
            h0 = r // 2
            lo, hi = out_ref[o:o + h0, :], out_ref[o + h0:o + r, :]
            s0[...] = jnp.where(lows[b][0], hi, lo)
            rdmas[b] = start(b * 6 + 0, b, 0, s0, rr0)
        for b, (o, r) in enumerate(SPLITS):
            acc1 = bufs[b][0]
            h0 = r // 2
            lo, hi = out_ref[o:o + h0, :], out_ref[o + h0:o + r, :]
            acc1[...] = jnp.where(lows[b][0], lo, hi)
        for b in range(3):
            rdmas[b].wait()
            bufs[b][0][...] += bufs[b][8][...]

        for b, (o, r) in enumerate(SPLITS):
            acc1, s1, rr1 = bufs[b][0], bufs[b][6], bufs[b][9]
            h1 = r // 4
            s1[...] = jnp.where(lows[b][1], acc1[h1:, :], acc1[:h1, :])
            rdmas[b] = start(b * 6 + 1, b, 1, s1, rr1)
        for b, (o, r) in enumerate(SPLITS):
            acc1, acc2 = bufs[b][0], bufs[b][1]
            h1 = r // 4
            acc2[...] = jnp.where(lows[b][1], acc1[:h1, :], acc1[h1:, :])
        for b in range(3):
            rdmas[b].wait()
            bufs[b][1][...] += bufs[b][9][...]

        for b, (o, r) in enumerate(SPLITS):
            acc2, s2, rr2 = bufs[b][1], bufs[b][7], bufs[b][10]
            h2 = r // 8
            s2[...] = jnp.where(lows[b][2], acc2[h2:, :], acc2[:h2, :])
            rdmas[b] = start(b * 6 + 2, b, 2, s2, rr2)
        for b, (o, r) in enumerate(SPLITS):
            acc2, g0 = bufs[b][1], bufs[b][2]
            h2 = r // 8
            g0[...] = jnp.where(lows[b][2], acc2[:h2, :], acc2[h2:, :])
        for b in range(3):
            rdmas[b].wait()
            bufs[b][2][...] += bufs[b][10][...]

        for b in range(3):
            g0, a0 = bufs[b][2], bufs[b][11]
            rdmas[b] = start(b * 6 + 3, b, 2, g0, a0)
        for b, (o, r) in enumerate(SPLITS):
            rdmas[b].wait()
            g0, g1, a0 = bufs[b][2], bufs[b][3], bufs[b][11]
            h2 = r // 8
            g1[:h2, :] = jnp.where(lows[b][2], g0[...], a0[...])
            g1[h2:, :] = jnp.where(lows[b][2], a0[...], g0[...])

        for b in range(3):
            g1, a1 = bufs[b][3], bufs[b][12]
            rdmas[b] = start(b * 6 + 4, b, 1, g1, a1)
        for b, (o, r) in enumerate(SPLITS):
            rdmas[b].wait()
            g1, g2, a1 = bufs[b][3], bufs[b][4], bufs[b][12]
            h1 = r // 4
            g2[:h1, :] = jnp.where(lows[b][1], g1[...], a1[...])
            g2[h1:, :] = jnp.where(lows[b][1], a1[...], g1[...])

        for b in range(3):
            g2, a2 = bufs[b][4], bufs[b][13]
            rdmas[b] = start(b * 6 + 5, b, 0, g2, a2)
        for b, (o, r) in enumerate(SPLITS):
            rdmas[b].wait()
            g2, a2 = bufs[b][4], bufs[b][13]
            h0 = r // 2
            out_ref[o:o + h0, :] = jnp.where(lows[b][0], g2[...], a2[...])
            out_ref[o + h0:o + r, :] = jnp.where(lows[b][0], a2[...], g2[...])

    scratch_shapes = []
    for _, r in SPLITS:
        scratch_shapes.extend(_block_scratch(r))
    scratch_shapes.append(pltpu.SemaphoreType.DMA((18,)))
    scratch_shapes.append(pltpu.SemaphoreType.DMA((18,)))

    return pl.pallas_call(
        body,
        out_shape=jax.ShapeDtypeStruct((M, M), jnp.float32),
        in_specs=[
            pl.BlockSpec(memory_space=pltpu.VMEM),
            pl.BlockSpec(memory_space=pltpu.VMEM),
        ],
        out_specs=pl.BlockSpec(memory_space=pltpu.VMEM),
        scratch_shapes=scratch_shapes,
        compiler_params=pltpu.CompilerParams(collective_id=0),
    )(dy, W)


# device time: 28652 ns/iter; 1.3917x vs baseline; 1.0031x over previous
import jax
import jax.numpy as jnp
from jax import lax
from jax.experimental import pallas as pl
from jax.experimental.pallas import tpu as pltpu

M = 512

SPLITS = ((0, 192), (192, 192), (384, 128))
RS_MASKS = ((4, 3, 1), (3, 1, 4), (1, 4, 3))


def kernel(dy, W):
    m, k = dy.shape
    assert W.shape == (m, k) and m == M

    def body(dy_ref, w_ref, out_ref, *scratch):
        send_sems, recv_sems = scratch[-2], scratch[-1]
        rrs = [scratch[b * 3:(b + 1) * 3] for b in range(3)]

        p = lax.axis_index("i")

        barrier_sem = pltpu.get_barrier_semaphore()
        for mask in (1, 3, 4):
            pl.semaphore_signal(
                barrier_sem, inc=1,
                device_id=(jnp.bitwise_xor(p, mask),),
                device_id_type=pl.DeviceIdType.MESH,
            )
        pl.semaphore_wait(barrier_sem, 3)

        out_ref[...] = lax.dot_general(
            dy_ref[...], w_ref[...],
            dimension_numbers=(((1,), (1,)), ((), ())),
            preferred_element_type=jnp.float32,
        )

        lows = [
            [(p & 4) == 0, (p & 2) == 0, (p & 1) == 0],
            [(p & 2) == 0, (p & 1) == 0, (p & 4) == 0],
            [((p ^ (p >> 1)) & 1) == 0, (p & 4) == 0, (p & 2) == 0],
        ]

        def start(e, b, s, src, dst):
            rdma = pltpu.make_async_remote_copy(
                src_ref=src, dst_ref=dst,
                send_sem=send_sems.at[e],
                recv_sem=recv_sems.at[e],
                device_id=(jnp.bitwise_xor(p, RS_MASKS[b][s]),),
                device_id_type=pl.DeviceIdType.MESH,
            )
            rdma.start()
            return rdma

        halves, keep_off, send_off = [], [], []
        for b, (o, r) in enumerate(SPLITS):
            hs = (r // 2, r // 4, r // 8)
            halves.append(hs)
            ds = [jnp.where(lows[b][s], 0, hs[s]) for s in range(3)]
            ko, so = [], []
            base = o
            for s in range(3):
                so.append(base + (hs[s] - ds[s]))
                base = base + ds[s]
                ko.append(base)
            keep_off.append(ko)
            send_off.append(so)

        rdmas = [None] * 3

        for s in range(3):
            for b in range(3):
                h = halves[b][s]
                rdmas[b] = start(
                    b * 6 + s, b, s,
                    out_ref.at[pl.ds(send_off[b][s], h)],
                    rrs[b][s],
                )
            for b in range(3):
                rdmas[b].wait()
                h = halves[b][s]
                out_ref[pl.ds(keep_off[b][s], h), :] = (
                    out_ref[pl.ds(keep_off[b][s], h), :] + rrs[b][s][...]
                )

        for t in range(3):
            s = 2 - t
            for b in range(3):
                h = halves[b][s]
                rdmas[b] = start(
                    b * 6 + 3 + t, b, s,
                    out_ref.at[pl.ds(keep_off[b][s], h)],
                    out_ref.at[pl.ds(keep_off[b][s], h)],
                )
            for b in range(3):
                rdmas[b].wait()

    scratch_shapes = []
    for _, r in SPLITS:
        for s in range(3):
            scratch_shapes.append(pltpu.VMEM((r >> (s + 1), M), jnp.float32))
    scratch_shapes.append(pltpu.SemaphoreType.DMA((18,)))
    scratch_shapes.append(pltpu.SemaphoreType.DMA((18,)))

    return pl.pallas_call(
        body,
        out_shape=jax.ShapeDtypeStruct((M, M), jnp.float32),
        in_specs=[
            pl.BlockSpec(memory_space=pltpu.VMEM),
            pl.BlockSpec(memory_space=pltpu.VMEM),
        ],
        out_specs=pl.BlockSpec(memory_space=pltpu.VMEM),
        scratch_shapes=scratch_shapes,
        compiler_params=pltpu.CompilerParams(collective_id=0),
    )(dy, W)


# device time: 26241 ns/iter; 1.5196x vs baseline; 1.0919x over previous
import jax
import jax.numpy as jnp
from jax import lax
from jax.experimental import pallas as pl
from jax.experimental.pallas import tpu as pltpu

M = 512
CW = 256

SPLITS = ((0, 192), (192, 192), (384, 128))
RS_MASKS = ((4, 3, 1), (3, 1, 4), (1, 4, 3))
CHUNKS = tuple((b, co) for co in (0, CW) for b in range(3))


def kernel(dy, W):
    m, k = dy.shape
    assert W.shape == (m, k) and m == M

    def body(dy_ref, w_ref, out_ref, *scratch):
        send_sems, recv_sems = scratch[-2], scratch[-1]
        rrs = [scratch[c * 3:(c + 1) * 3] for c in range(len(CHUNKS))]

        p = lax.axis_index("i")

        barrier_sem = pltpu.get_barrier_semaphore()
        for mask in (1, 3, 4):
            pl.semaphore_signal(
                barrier_sem, inc=1,
                device_id=(jnp.bitwise_xor(p, mask),),
                device_id_type=pl.DeviceIdType.MESH,
            )
        pl.semaphore_wait(barrier_sem, 3)

        out_ref[...] = lax.dot_general(
            dy_ref[...], w_ref[...],
            dimension_numbers=(((1,), (1,)), ((), ())),
            preferred_element_type=jnp.float32,
        )

        lows = [
            [(p & 4) == 0, (p & 2) == 0, (p & 1) == 0],
            [(p & 2) == 0, (p & 1) == 0, (p & 4) == 0],
            [((p ^ (p >> 1)) & 1) == 0, (p & 4) == 0, (p & 2) == 0],
        ]

        halves, keep_off, send_off = [], [], []
        for b, (o, r) in enumerate(SPLITS):
            hs = (r // 2, r // 4, r // 8)
            halves.append(hs)
            ds = [jnp.where(lows[b][s], 0, hs[s]) for s in range(3)]
            ko, so = [], []
            base = o
            for s in range(3):
                so.append(base + (hs[s] - ds[s]))
                base = base + ds[s]
                ko.append(base)
            keep_off.append(ko)
            send_off.append(so)

        def start(c, step):
            b, co = CHUNKS[c]
            s = step if step < 3 else 5 - step
            h = halves[b][s]
            if step < 3:
                src = out_ref.at[pl.ds(send_off[b][s], h), pl.ds(co, CW)]
                dst = rrs[c][s]
            else:
                src = out_ref.at[pl.ds(keep_off[b][s], h), pl.ds(co, CW)]
                dst = src
            rdma = pltpu.make_async_remote_copy(
                src_ref=src, dst_ref=dst,
                send_sem=send_sems.at[c * 6 + step],
                recv_sem=recv_sems.at[c * 6 + step],
                device_id=(jnp.bitwise_xor(p, RS_MASKS[b][s]),),
                device_id_type=pl.DeviceIdType.MESH,
            )
            rdma.start()
            return rdma

        n = len(CHUNKS)
        rdmas = [None] * n
        for step in range(6):
            for c in range(n):
                b, co = CHUNKS[c]
                if step > 0:
                    rdmas[c].wait()
                    if step <= 3:
                        s = step - 1
                        h = halves[b][s]
                        out_ref[pl.ds(keep_off[b][s], h), co:co + CW] = (
                            out_ref[pl.ds(keep_off[b][s], h), co:co + CW]
                            + rrs[c][s][...]
                        )
                rdmas[c] = start(c, step)
        for c in range(n):
            rdmas[c].wait()

    scratch_shapes = []
    for b, _ in CHUNKS:
        r = SPLITS[b][1]
        for s in range(3):
            scratch_shapes.append(
                pltpu.VMEM((r >> (s + 1), CW), jnp.float32)
            )
    scratch_shapes.append(pltpu.SemaphoreType.DMA((6 * len(CHUNKS),)))
    scratch_shapes.append(pltpu.SemaphoreType.DMA((6 * len(CHUNKS),)))

    return pl.pallas_call(
        body,
        out_shape=jax.ShapeDtypeStruct((M, M), jnp.float32),
        in_specs=[
            pl.BlockSpec(memory_space=pltpu.VMEM),
            pl.BlockSpec(memory_space=pltpu.VMEM),
        ],
        out_specs=pl.BlockSpec(memory_space=pltpu.VMEM),
        scratch_shapes=scratch_shapes,
        compiler_params=pltpu.CompilerParams(collective_id=0),
    )(dy, W)
